# Initial kernel scaffold; baseline (speedup 1.0000x reference)
import jax
import jax.numpy as jnp
from jax.experimental import pallas as pl


def kernel(x, edge_index, P_vec, W1, b1, W2, b2, W3, b3, index):
    raise NotImplementedError("write your pallas kernel here")


# speedup vs baseline: 7.1406x; 7.1406x over previous
import jax
import jax.numpy as jnp
from jax import lax
from jax.experimental import pallas as pl
from jax.experimental.pallas import tpu as pltpu
from jax.experimental.pallas import tpu_sc as plsc

N = 10000
NP = 10240
E = 160000
D = 256
C = 16
NC, NS, L = 2, 16, 16
NW = NC * NS
EP = 163840
EW_T = EP // NS
K = 128
H = 128
ROWS_T = NP // NS

_mesh = plsc.VectorSubcoreMesh(
    core_axis_name="c", subcore_axis_name="s", num_cores=NC, num_subcores=NS)

def _bcast(vec, lane):
    return jnp.full((L,), vec[lane], jnp.float32)


def _zero2d(ref, nrows, ncols):
    z = jnp.zeros((L,), jnp.float32)

    def body(i, _):
        r = i // (ncols // L)
        k = i % (ncols // L)
        ref[r, pl.ds(k * L, L)] = z
        return 0

    lax.fori_loop(0, nrows * (ncols // L), body, 0)


def _deg_body(p_hbm, src_hbm, dst_hbm, idx_hbm, ew_hbm, du_hbm,
              p_v, idx_v, dval_v, i16_v, rows_v, acc_sh):
    c = lax.axis_index("c")
    s = lax.axis_index("s")
    base = s * EW_T

    _zero2d(rows_v, K, H)
    for q in range(5):
        pltpu.sync_copy(rows_v, acc_sh.at[pl.ds(s * ROWS_T + q * K, K)])

    pltpu.sync_copy(p_hbm.at[pl.ds(base, EW_T)], p_v)
    pltpu.sync_copy(dst_hbm.at[pl.ds(base, EW_T)], dval_v)

    @pl.when(c == 0)
    def _():
        pltpu.sync_copy(dst_hbm.at[pl.ds(base, EW_T)], idx_v)

    @pl.when(c == 1)
    def _():
        pltpu.sync_copy(src_hbm.at[pl.ds(base, EW_T)], idx_v)

    pltpu.sync_copy(idx_hbm, i16_v)
    plsc.subcore_barrier()
    tgt = i16_v[...]

    iota = lax.iota(jnp.int32, L)
    cf = jnp.full((L,), c, jnp.int32).astype(jnp.float32)

    def group(g, _):
        pv = p_v[pl.ds(g * L, L)]
        ew = 1.0 / (1.0 + jnp.exp(-pv))
        gid = base + g * L + iota
        ew = jnp.where(gid < E, ew, 0.0)
        d16 = dval_v[pl.ds(g * L, L)]
        wm = jnp.where(d16 == tgt, ew, 0.0)
        p_v[pl.ds(g * L, L)] = ew * (1.0 - cf) + wm * cf
        return 0

    lax.fori_loop(0, EW_T // L, group, 0)

    def chunk(j, _):
        def grp(g, _):
            w16v = p_v[pl.ds(j * K + g * L, L)]
            for t in range(L):
                wb = _bcast(w16v, t)
                r = g * L + t
                for k in range(H // L):
                    rows_v[r, pl.ds(k * L, L)] = wb
            return 0

        lax.fori_loop(0, K // L, grp, 0)
        pltpu.sync_copy(rows_v, acc_sh.at[idx_v.at[pl.ds(j * K, K)]], add=True)
        return 0

    lax.fori_loop(0, EW_T // K, chunk, 0)

    @pl.when(c == 0)
    def _():
        pltpu.sync_copy(p_v, ew_hbm.at[pl.ds(base, EW_T)])

    plsc.subcore_barrier()
    pltpu.sync_copy(acc_sh.at[pl.ds(s * ROWS_T, ROWS_T)],
                    du_hbm.at[c].at[pl.ds(s * ROWS_T, ROWS_T)])


def _deg_kernel(p_pad, src_p, dst_p, idx16):
    return pl.kernel(
        _deg_body,
        out_type=(jax.ShapeDtypeStruct((EP,), jnp.float32),
                  jax.ShapeDtypeStruct((NC, NP, H), jnp.float32)),
        mesh=_mesh,
        scratch_types=[
            pltpu.VMEM((EW_T,), jnp.float32),
            pltpu.VMEM((EW_T,), jnp.int32),
            pltpu.VMEM((EW_T,), jnp.int32),
            pltpu.VMEM((L,), jnp.int32),
            pltpu.VMEM((K, H), jnp.float32),
            pltpu.VMEM_SHARED((NP, H), jnp.float32),
        ],
    )(p_pad, src_p, dst_p, idx16)


def _agg_body(gp_hbm, src_hbm, ew_hbm, dst_hbm, agg_hbm,
              src_v, ew_v, dst_v, rows_v, acc_sh, sem):
    c = lax.axis_index("c")
    s = lax.axis_index("s")
    base = s * EW_T

    _zero2d(rows_v, K, H)
    for q in range(5):
        pltpu.sync_copy(rows_v, acc_sh.at[pl.ds(s * ROWS_T + q * K, K)])

    pltpu.sync_copy(src_hbm.at[pl.ds(base, EW_T)], src_v)
    pltpu.sync_copy(ew_hbm.at[pl.ds(base, EW_T)], ew_v)
    pltpu.sync_copy(dst_hbm.at[pl.ds(base, EW_T)], dst_v)
    plsc.subcore_barrier()

    def chunk(j, _):
        pltpu.async_copy(gp_hbm.at[c].at[src_v.at[pl.ds(j * K, K)]],
                         rows_v, sem).wait()

        def grp(g, _):
            ew16 = ew_v[pl.ds(j * K + g * L, L)]
            for t in range(L):
                w16 = _bcast(ew16, t)
                r = g * L + t
                for k in range(H // L):
                    rows_v[r, pl.ds(k * L, L)] = rows_v[r, pl.ds(k * L, L)] * w16
            return 0

        lax.fori_loop(0, K // L, grp, 0)
        pltpu.sync_copy(rows_v, acc_sh.at[dst_v.at[pl.ds(j * K, K)]], add=True)
        return 0

    lax.fori_loop(0, EW_T // K, chunk, 0)

    plsc.subcore_barrier()
    pltpu.sync_copy(acc_sh.at[pl.ds(s * ROWS_T, ROWS_T)],
                    agg_hbm.at[c].at[pl.ds(s * ROWS_T, ROWS_T)])


def _agg_kernel(gp, src_p, ew_p, dst_p):
    return pl.kernel(
        _agg_body,
        out_type=jax.ShapeDtypeStruct((NC, NP, H), jnp.float32),
        mesh=_mesh,
        scratch_types=[
            pltpu.VMEM((EW_T,), jnp.int32),
            pltpu.VMEM((EW_T,), jnp.float32),
            pltpu.VMEM((EW_T,), jnp.int32),
            pltpu.VMEM((K, H), jnp.float32),
            pltpu.VMEM_SHARED((NP, H), jnp.float32),
            pltpu.SemaphoreType.DMA,
        ],
    )(gp, src_p, ew_p, dst_p)


BR = 2048


def _tcA_body(x_ref, w_ref, du_ref, out_ref):
    deg = 1.0 + du_ref[0, :, 0]
    dinv = lax.rsqrt(deg)
    g = jnp.dot(x_ref[...], w_ref[...], preferred_element_type=jnp.float32)
    gp = g * dinv[:, None]
    out_ref[0] = gp[:, :H]
    out_ref[1] = gp[:, H:]


def _tcA(x, w1, du):
    return pl.pallas_call(
        _tcA_body,
        grid=(NP // BR,),
        in_specs=[
            pl.BlockSpec((BR, D), lambda i: (i, 0)),
            pl.BlockSpec((D, D), lambda i: (0, 0)),
            pl.BlockSpec((NC, BR, H), lambda i: (0, i, 0)),
        ],
        out_specs=pl.BlockSpec((NC, BR, H), lambda i: (0, i, 0)),
        out_shape=jax.ShapeDtypeStruct((NC, NP, H), jnp.float32),
    )(x, w1, du)


def _tcB_body(acc_ref, gp_ref, du_ref, b_ref, w_ref, out_ref):
    dinv = lax.rsqrt(1.0 + du_ref[0, :, 0])
    lo = dinv[:, None] * (acc_ref[0] + gp_ref[0])
    hi = dinv[:, None] * (acc_ref[1] + gp_ref[1])
    h = jnp.concatenate([lo, hi], axis=1) + b_ref[0][None, :]
    h = jnp.maximum(h, 0.0)
    g = jnp.dot(h, w_ref[...], preferred_element_type=jnp.float32)
    gp = g * dinv[:, None]
    out_ref[0] = gp[:, :H]
    out_ref[1] = gp[:, H:]


def _tcB(acc, gp, du, b, w):
    return pl.pallas_call(
        _tcB_body,
        grid=(NP // BR,),
        in_specs=[
            pl.BlockSpec((NC, BR, H), lambda i: (0, i, 0)),
            pl.BlockSpec((NC, BR, H), lambda i: (0, i, 0)),
            pl.BlockSpec((NC, BR, H), lambda i: (0, i, 0)),
            pl.BlockSpec((1, D), lambda i: (0, 0)),
            pl.BlockSpec((D, D), lambda i: (0, 0)),
        ],
        out_specs=pl.BlockSpec((NC, BR, H), lambda i: (0, i, 0)),
        out_shape=jax.ShapeDtypeStruct((NC, NP, H), jnp.float32),
    )(acc, gp, du, b.reshape(1, D), w)


def _tcC_body(acc_ref, gp_ref, du_ref, b_ref, w_ref, out_ref, a3_ref):
    i = pl.program_id(0)
    dinv = lax.rsqrt(1.0 + du_ref[0, :, 0])
    u = du_ref[1, :, 0]
    lo = dinv[:, None] * (acc_ref[0] + gp_ref[0])
    hi = dinv[:, None] * (acc_ref[1] + gp_ref[1])
    h = jnp.concatenate([lo, hi], axis=1) + b_ref[0][None, :]
    h = jnp.maximum(h, 0.0)
    g = jnp.dot(h, w_ref[...], preferred_element_type=jnp.float32)
    gp3 = g * dinv[:, None]
    out_ref[...] = gp3
    part = jnp.dot(u[None, :], gp3, preferred_element_type=jnp.float32)

    @pl.when(i == 0)
    def _():
        a3_ref[...] = jnp.zeros_like(a3_ref)

    a3_ref[...] += part


def _tcC(acc, gp, du, b, w3):
    return pl.pallas_call(
        _tcC_body,
        grid=(NP // BR,),
        in_specs=[
            pl.BlockSpec((NC, BR, H), lambda i: (0, i, 0)),
            pl.BlockSpec((NC, BR, H), lambda i: (0, i, 0)),
            pl.BlockSpec((NC, BR, H), lambda i: (0, i, 0)),
            pl.BlockSpec((1, D), lambda i: (0, 0)),
            pl.BlockSpec((D, C), lambda i: (0, 0)),
        ],
        out_specs=[
            pl.BlockSpec((BR, C), lambda i: (i, 0)),
            pl.BlockSpec((1, C), lambda i: (0, 0)),
        ],
        out_shape=[jax.ShapeDtypeStruct((NP, C), jnp.float32),
                   jax.ShapeDtypeStruct((1, C), jnp.float32)],
    )(acc, gp, du, b.reshape(1, D), w3)


def kernel(x, edge_index, P_vec, W1, b1, W2, b2, W3, b3, index):
    i32 = jnp.int32
    src = edge_index[0].astype(i32)
    dst = edge_index[1].astype(i32)
    pad = EP - E
    src_p = jnp.concatenate([src, jnp.zeros((pad,), i32)])
    dst_p = jnp.concatenate([dst, jnp.zeros((pad,), i32)])
    p_pad = jnp.concatenate([P_vec, jnp.zeros((pad,), jnp.float32)])
    x_p = jnp.concatenate([x, jnp.zeros((NP - N, D), jnp.float32)])
    idx16 = jnp.full((L,), index, i32)

    ew_p, du = _deg_kernel(p_pad, src_p, dst_p, idx16)
    gp1 = _tcA(x_p, W1, du)
    acc1 = _agg_kernel(gp1, src_p, ew_p, dst_p)
    gp2 = _tcB(acc1, gp1, du, b1, W2)
    acc2 = _agg_kernel(gp2, src_p, ew_p, dst_p)
    gp3, acc3 = _tcC(acc2, gp2, du, b2, W3)

    dinv_i = lax.rsqrt(1.0 + du[0, index, 0])
    row = dinv_i * (acc3[0] + gp3[index]) + b3
    m = jnp.max(row)
    logp = row - (m + jnp.log(jnp.sum(jnp.exp(row - m))))
    return logp


# speedup vs baseline: 8.4323x; 1.1809x over previous
import jax
import jax.numpy as jnp
from jax import lax
from jax.experimental import pallas as pl
from jax.experimental.pallas import tpu as pltpu
from jax.experimental.pallas import tpu_sc as plsc

N = 10000
NP = 10240
E = 160000
D = 256
C = 16
NC, NS, L = 2, 16, 16
NW = NC * NS
EP = 163840
EW_T = EP // NS
K = 128
KC = 64
NCH = EW_T // KC
H = 128
ROWS_T = NP // NS

_mesh = plsc.VectorSubcoreMesh(
    core_axis_name="c", subcore_axis_name="s", num_cores=NC, num_subcores=NS)

def _bcast(vec, lane):
    return jnp.full((L,), vec[lane], jnp.float32)


def _zero2d(ref, nrows, ncols):
    z = jnp.zeros((L,), jnp.float32)

    def body(i, _):
        r = i // (ncols // L)
        k = i % (ncols // L)
        ref[r, pl.ds(k * L, L)] = z
        return 0

    lax.fori_loop(0, nrows * (ncols // L), body, 0)


def _deg_body(p_hbm, src_hbm, dst_hbm, idx_hbm, ew_hbm, du_hbm,
              p_v, idx_v, dval_v, i16_v, rows0_v, rows1_v, acc_sh,
              ssem0, ssem1):
    c = lax.axis_index("c")
    s = lax.axis_index("s")
    base = s * EW_T

    _zero2d(rows0_v, KC, H)
    for q in range(ROWS_T // KC):
        pltpu.sync_copy(rows0_v, acc_sh.at[pl.ds(s * ROWS_T + q * KC, KC)])

    pltpu.sync_copy(p_hbm.at[pl.ds(base, EW_T)], p_v)
    pltpu.sync_copy(dst_hbm.at[pl.ds(base, EW_T)], dval_v)

    @pl.when(c == 0)
    def _():
        pltpu.sync_copy(dst_hbm.at[pl.ds(base, EW_T)], idx_v)

    @pl.when(c == 1)
    def _():
        pltpu.sync_copy(src_hbm.at[pl.ds(base, EW_T)], idx_v)

    pltpu.sync_copy(idx_hbm, i16_v)
    plsc.subcore_barrier()
    tgt = i16_v[...]

    iota = lax.iota(jnp.int32, L)
    cf = jnp.full((L,), c, jnp.int32).astype(jnp.float32)

    def group(g, _):
        pv = p_v[pl.ds(g * L, L)]
        ew = 1.0 / (1.0 + jnp.exp(-pv))
        gid = base + g * L + iota
        ew = jnp.where(gid < E, ew, 0.0)
        d16 = dval_v[pl.ds(g * L, L)]
        wm = jnp.where(d16 == tgt, ew, 0.0)
        p_v[pl.ds(g * L, L)] = ew * (1.0 - cf) + wm * cf
        return 0

    lax.fori_loop(0, EW_T // L, group, 0)

    bufs = (rows0_v, rows1_v)
    ssems = (ssem0, ssem1)

    def sdst(jj):
        return acc_sh.at[idx_v.at[pl.ds(jj * KC, KC)]]

    def build(buf, jj):
        def grp(g, _):
            w16v = p_v[pl.ds(jj * KC + g * L, L)]
            for t in range(L):
                wb = _bcast(w16v, t)
                r = g * L + t
                for k in range(H // L):
                    buf[r, pl.ds(k * L, L)] = wb
            return 0

        lax.fori_loop(0, KC // L, grp, 0)

    def half(X, jj):
        bX = bufs[X]

        @pl.when(jj >= 2)
        def _():
            pltpu.make_async_copy(bX, sdst(jj - 2), ssems[X]).wait()

        build(bX, jj)
        pltpu.async_copy(bX, sdst(jj), ssems[X], add=True)

    def piped(j2, _):
        half(0, 2 * j2)
        half(1, 2 * j2 + 1)
        return 0

    lax.fori_loop(0, NCH // 2, piped, 0)
    pltpu.make_async_copy(rows0_v, sdst(NCH - 2), ssem0).wait()
    pltpu.make_async_copy(rows1_v, sdst(NCH - 1), ssem1).wait()

    @pl.when(c == 0)
    def _():
        pltpu.sync_copy(p_v, ew_hbm.at[pl.ds(base, EW_T)])

    plsc.subcore_barrier()
    pltpu.sync_copy(acc_sh.at[pl.ds(s * ROWS_T, ROWS_T)],
                    du_hbm.at[c].at[pl.ds(s * ROWS_T, ROWS_T)])


def _deg_kernel(p_pad, src_p, dst_p, idx16):
    return pl.kernel(
        _deg_body,
        out_type=(jax.ShapeDtypeStruct((EP,), jnp.float32),
                  jax.ShapeDtypeStruct((NC, NP, H), jnp.float32)),
        mesh=_mesh,
        scratch_types=[
            pltpu.VMEM((EW_T,), jnp.float32),
            pltpu.VMEM((EW_T,), jnp.int32),
            pltpu.VMEM((EW_T,), jnp.int32),
            pltpu.VMEM((L,), jnp.int32),
            pltpu.VMEM((KC, H), jnp.float32),
            pltpu.VMEM((KC, H), jnp.float32),
            pltpu.VMEM_SHARED((NP, H), jnp.float32),
            pltpu.SemaphoreType.DMA,
            pltpu.SemaphoreType.DMA,
        ],
    )(p_pad, src_p, dst_p, idx16)


def _agg_body(gp_hbm, src_hbm, ew_hbm, dst_hbm, agg_hbm,
              src_v, ew_v, dst_v, rows0_v, rows1_v, acc_sh,
              gsem0, gsem1, ssem0, ssem1):
    c = lax.axis_index("c")
    s = lax.axis_index("s")
    base = s * EW_T

    _zero2d(rows0_v, KC, H)
    for q in range(ROWS_T // KC):
        pltpu.sync_copy(rows0_v, acc_sh.at[pl.ds(s * ROWS_T + q * KC, KC)])

    pltpu.sync_copy(src_hbm.at[pl.ds(base, EW_T)], src_v)
    pltpu.sync_copy(ew_hbm.at[pl.ds(base, EW_T)], ew_v)
    pltpu.sync_copy(dst_hbm.at[pl.ds(base, EW_T)], dst_v)
    plsc.subcore_barrier()

    bufs = (rows0_v, rows1_v)
    gsems = (gsem0, gsem1)
    ssems = (ssem0, ssem1)

    def gsrc(jj):
        return gp_hbm.at[c].at[src_v.at[pl.ds(jj * KC, KC)]]

    def sdst(jj):
        return acc_sh.at[dst_v.at[pl.ds(jj * KC, KC)]]

    def scale(buf, jj):
        def grp(g, _):
            ew16 = ew_v[pl.ds(jj * KC + g * L, L)]
            for t in range(L):
                w16 = _bcast(ew16, t)
                r = g * L + t
                for k in range(H // L):
                    buf[r, pl.ds(k * L, L)] = buf[r, pl.ds(k * L, L)] * w16
            return 0

        lax.fori_loop(0, KC // L, grp, 0)

    def half(X, jj):
        bX, bY = bufs[X], bufs[1 - X]
        pltpu.make_async_copy(gsrc(jj), bX, gsems[X]).wait()

        @pl.when(jj >= 1)
        def _():
            pltpu.make_async_copy(bY, sdst(jj - 1), ssems[1 - X]).wait()

        @pl.when(jj + 1 < NCH)
        def _():
            pltpu.async_copy(gsrc(jj + 1), bY, gsems[1 - X])

        scale(bX, jj)
        pltpu.async_copy(bX, sdst(jj), ssems[X], add=True)

    pltpu.async_copy(gsrc(0), rows0_v, gsem0)

    def piped(j2, _):
        half(0, 2 * j2)
        half(1, 2 * j2 + 1)
        return 0

    lax.fori_loop(0, NCH // 2, piped, 0)
    pltpu.make_async_copy(rows1_v, sdst(NCH - 1), ssem1).wait()

    plsc.subcore_barrier()
    pltpu.sync_copy(acc_sh.at[pl.ds(s * ROWS_T, ROWS_T)],
                    agg_hbm.at[c].at[pl.ds(s * ROWS_T, ROWS_T)])


def _agg_kernel(gp, src_p, ew_p, dst_p):
    return pl.kernel(
        _agg_body,
        out_type=jax.ShapeDtypeStruct((NC, NP, H), jnp.float32),
        mesh=_mesh,
        scratch_types=[
            pltpu.VMEM((EW_T,), jnp.int32),
            pltpu.VMEM((EW_T,), jnp.float32),
            pltpu.VMEM((EW_T,), jnp.int32),
            pltpu.VMEM((KC, H), jnp.float32),
            pltpu.VMEM((KC, H), jnp.float32),
            pltpu.VMEM_SHARED((NP, H), jnp.float32),
            pltpu.SemaphoreType.DMA,
            pltpu.SemaphoreType.DMA,
            pltpu.SemaphoreType.DMA,
            pltpu.SemaphoreType.DMA,
        ],
    )(gp, src_p, ew_p, dst_p)


BR = 2048


def _tcA_body(x_ref, w_ref, du_ref, out_ref):
    deg = 1.0 + du_ref[0, :, 0]
    dinv = lax.rsqrt(deg)
    g = jnp.dot(x_ref[...], w_ref[...], preferred_element_type=jnp.float32)
    gp = g * dinv[:, None]
    out_ref[0] = gp[:, :H]
    out_ref[1] = gp[:, H:]


def _tcA(x, w1, du):
    return pl.pallas_call(
        _tcA_body,
        grid=(NP // BR,),
        in_specs=[
            pl.BlockSpec((BR, D), lambda i: (i, 0)),
            pl.BlockSpec((D, D), lambda i: (0, 0)),
            pl.BlockSpec((NC, BR, H), lambda i: (0, i, 0)),
        ],
        out_specs=pl.BlockSpec((NC, BR, H), lambda i: (0, i, 0)),
        out_shape=jax.ShapeDtypeStruct((NC, NP, H), jnp.float32),
    )(x, w1, du)


def _tcB_body(acc_ref, gp_ref, du_ref, b_ref, w_ref, out_ref):
    dinv = lax.rsqrt(1.0 + du_ref[0, :, 0])
    lo = dinv[:, None] * (acc_ref[0] + gp_ref[0])
    hi = dinv[:, None] * (acc_ref[1] + gp_ref[1])
    h = jnp.concatenate([lo, hi], axis=1) + b_ref[0][None, :]
    h = jnp.maximum(h, 0.0)
    g = jnp.dot(h, w_ref[...], preferred_element_type=jnp.float32)
    gp = g * dinv[:, None]
    out_ref[0] = gp[:, :H]
    out_ref[1] = gp[:, H:]


def _tcB(acc, gp, du, b, w):
    return pl.pallas_call(
        _tcB_body,
        grid=(NP // BR,),
        in_specs=[
            pl.BlockSpec((NC, BR, H), lambda i: (0, i, 0)),
            pl.BlockSpec((NC, BR, H), lambda i: (0, i, 0)),
            pl.BlockSpec((NC, BR, H), lambda i: (0, i, 0)),
            pl.BlockSpec((1, D), lambda i: (0, 0)),
            pl.BlockSpec((D, D), lambda i: (0, 0)),
        ],
        out_specs=pl.BlockSpec((NC, BR, H), lambda i: (0, i, 0)),
        out_shape=jax.ShapeDtypeStruct((NC, NP, H), jnp.float32),
    )(acc, gp, du, b.reshape(1, D), w)


def _tcC_body(acc_ref, gp_ref, du_ref, b_ref, w_ref, out_ref, a3_ref):
    i = pl.program_id(0)
    dinv = lax.rsqrt(1.0 + du_ref[0, :, 0])
    u = du_ref[1, :, 0]
    lo = dinv[:, None] * (acc_ref[0] + gp_ref[0])
    hi = dinv[:, None] * (acc_ref[1] + gp_ref[1])
    h = jnp.concatenate([lo, hi], axis=1) + b_ref[0][None, :]
    h = jnp.maximum(h, 0.0)
    g = jnp.dot(h, w_ref[...], preferred_element_type=jnp.float32)
    gp3 = g * dinv[:, None]
    out_ref[...] = gp3
    part = jnp.dot(u[None, :], gp3, preferred_element_type=jnp.float32)

    @pl.when(i == 0)
    def _():
        a3_ref[...] = jnp.zeros_like(a3_ref)

    a3_ref[...] += part


def _tcC(acc, gp, du, b, w3):
    return pl.pallas_call(
        _tcC_body,
        grid=(NP // BR,),
        in_specs=[
            pl.BlockSpec((NC, BR, H), lambda i: (0, i, 0)),
            pl.BlockSpec((NC, BR, H), lambda i: (0, i, 0)),
            pl.BlockSpec((NC, BR, H), lambda i: (0, i, 0)),
            pl.BlockSpec((1, D), lambda i: (0, 0)),
            pl.BlockSpec((D, C), lambda i: (0, 0)),
        ],
        out_specs=[
            pl.BlockSpec((BR, C), lambda i: (i, 0)),
            pl.BlockSpec((1, C), lambda i: (0, 0)),
        ],
        out_shape=[jax.ShapeDtypeStruct((NP, C), jnp.float32),
                   jax.ShapeDtypeStruct((1, C), jnp.float32)],
    )(acc, gp, du, b.reshape(1, D), w3)


def kernel(x, edge_index, P_vec, W1, b1, W2, b2, W3, b3, index):
    i32 = jnp.int32
    src = edge_index[0].astype(i32)
    dst = edge_index[1].astype(i32)
    pad = EP - E
    src_p = jnp.concatenate([src, jnp.zeros((pad,), i32)])
    dst_p = jnp.concatenate([dst, jnp.zeros((pad,), i32)])
    p_pad = jnp.concatenate([P_vec, jnp.zeros((pad,), jnp.float32)])
    x_p = jnp.concatenate([x, jnp.zeros((NP - N, D), jnp.float32)])
    idx16 = jnp.full((L,), index, i32)

    ew_p, du = _deg_kernel(p_pad, src_p, dst_p, idx16)
    gp1 = _tcA(x_p, W1, du)
    acc1 = _agg_kernel(gp1, src_p, ew_p, dst_p)
    gp2 = _tcB(acc1, gp1, du, b1, W2)
    acc2 = _agg_kernel(gp2, src_p, ew_p, dst_p)
    gp3, acc3 = _tcC(acc2, gp2, du, b2, W3)

    dinv_i = lax.rsqrt(1.0 + du[0, index, 0])
    row = dinv_i * (acc3[0] + gp3[index]) + b3
    m = jnp.max(row)
    logp = row - (m + jnp.log(jnp.sum(jnp.exp(row - m))))
    return logp


# speedup vs baseline: 9.3522x; 1.1091x over previous
import jax
import jax.numpy as jnp
from jax import lax
from jax.experimental import pallas as pl
from jax.experimental.pallas import tpu as pltpu
from jax.experimental.pallas import tpu_sc as plsc

N = 10000
NP = 10240
E = 160000
D = 256
C = 16
NC, NS, L = 2, 16, 16
NW = NC * NS
EP = 163840
EW_T = EP // NS
K = 128
KC = 64
NCH = EW_T // KC
KA = 32
NCA = EW_T // KA
H = 128
ROWS_T = NP // NS

_mesh = plsc.VectorSubcoreMesh(
    core_axis_name="c", subcore_axis_name="s", num_cores=NC, num_subcores=NS)

def _bcast(vec, lane):
    return jnp.full((L,), vec[lane], jnp.float32)


def _zero2d(ref, nrows, ncols):
    z = jnp.zeros((L,), jnp.float32)

    def body(i, _):
        r = i // (ncols // L)
        k = i % (ncols // L)
        ref[r, pl.ds(k * L, L)] = z
        return 0

    lax.fori_loop(0, nrows * (ncols // L), body, 0)


def _deg_body(p_hbm, src_hbm, dst_hbm, idx_hbm, ew_hbm, du_hbm,
              p_v, idx_v, dval_v, i16_v, rows0_v, rows1_v, acc_sh,
              ssem0, ssem1):
    c = lax.axis_index("c")
    s = lax.axis_index("s")
    base = s * EW_T

    _zero2d(rows0_v, KC, H)
    for q in range(ROWS_T // KC):
        pltpu.async_copy(rows0_v, acc_sh.at[pl.ds(s * ROWS_T + q * KC, KC)], ssem0)
    pltpu.async_copy(p_hbm.at[pl.ds(base, EW_T)], p_v, ssem1)
    pltpu.async_copy(dst_hbm.at[pl.ds(base, EW_T)], dval_v, ssem1)

    @pl.when(c == 0)
    def _():
        pltpu.sync_copy(dst_hbm.at[pl.ds(base, EW_T)], idx_v)

    @pl.when(c == 1)
    def _():
        pltpu.sync_copy(src_hbm.at[pl.ds(base, EW_T)], idx_v)

    pltpu.sync_copy(idx_hbm, i16_v)
    for q in range(ROWS_T // KC):
        pltpu.make_async_copy(rows0_v, acc_sh.at[pl.ds(s * ROWS_T + q * KC, KC)],
                              ssem0).wait()
    pltpu.make_async_copy(p_hbm.at[pl.ds(base, EW_T)], p_v, ssem1).wait()
    pltpu.make_async_copy(dst_hbm.at[pl.ds(base, EW_T)], dval_v, ssem1).wait()
    plsc.subcore_barrier()
    tgt = i16_v[...]

    iota = lax.iota(jnp.int32, L)
    cf = jnp.full((L,), c, jnp.int32).astype(jnp.float32)

    def group(g, _):
        pv = p_v[pl.ds(g * L, L)]
        ew = 1.0 / (1.0 + jnp.exp(-pv))
        gid = base + g * L + iota
        ew = jnp.where(gid < E, ew, 0.0)
        d16 = dval_v[pl.ds(g * L, L)]
        wm = jnp.where(d16 == tgt, ew, 0.0)
        p_v[pl.ds(g * L, L)] = ew * (1.0 - cf) + wm * cf
        return 0

    lax.fori_loop(0, EW_T // L, group, 0)

    bufs = (rows0_v, rows1_v)
    ssems = (ssem0, ssem1)

    def sdst(jj):
        return acc_sh.at[idx_v.at[pl.ds(jj * KC, KC)]]

    def build(buf, jj):
        def grp(g, _):
            w16v = p_v[pl.ds(jj * KC + g * L, L)]
            for t in range(L):
                wb = _bcast(w16v, t)
                r = g * L + t
                for k in range(H // L):
                    buf[r, pl.ds(k * L, L)] = wb
            return 0

        lax.fori_loop(0, KC // L, grp, 0)

    def half(X, jj):
        bX = bufs[X]

        @pl.when(jj >= 2)
        def _():
            pltpu.make_async_copy(bX, sdst(jj - 2), ssems[X]).wait()

        build(bX, jj)
        pltpu.async_copy(bX, sdst(jj), ssems[X], add=True)

    def piped(j2, _):
        half(0, 2 * j2)
        half(1, 2 * j2 + 1)
        return 0

    lax.fori_loop(0, NCH // 2, piped, 0)
    pltpu.make_async_copy(rows0_v, sdst(NCH - 2), ssem0).wait()
    pltpu.make_async_copy(rows1_v, sdst(NCH - 1), ssem1).wait()

    @pl.when(c == 0)
    def _():
        pltpu.sync_copy(p_v, ew_hbm.at[pl.ds(base, EW_T)])

    plsc.subcore_barrier()
    pltpu.sync_copy(acc_sh.at[pl.ds(s * ROWS_T, ROWS_T)],
                    du_hbm.at[c].at[pl.ds(s * ROWS_T, ROWS_T)])


def _deg_kernel(p_pad, src_p, dst_p, idx16):
    return pl.kernel(
        _deg_body,
        out_type=(jax.ShapeDtypeStruct((EP,), jnp.float32),
                  jax.ShapeDtypeStruct((NC, NP, H), jnp.float32)),
        mesh=_mesh,
        scratch_types=[
            pltpu.VMEM((EW_T,), jnp.float32),
            pltpu.VMEM((EW_T,), jnp.int32),
            pltpu.VMEM((EW_T,), jnp.int32),
            pltpu.VMEM((L,), jnp.int32),
            pltpu.VMEM((KC, H), jnp.float32),
            pltpu.VMEM((KC, H), jnp.float32),
            pltpu.VMEM_SHARED((NP, H), jnp.float32),
            pltpu.SemaphoreType.DMA,
            pltpu.SemaphoreType.DMA,
        ],
    )(p_pad, src_p, dst_p, idx16)


def _agg_body(gp_hbm, src_hbm, ew_hbm, dst_hbm, agg_hbm,
              src_v, ew_v, dst_v, r0, r1, r2, r3, acc_sh,
              g0, g1, g2, g3, s0, s1, s2, s3):
    c = lax.axis_index("c")
    s = lax.axis_index("s")
    base = s * EW_T

    bufs = (r0, r1, r2, r3)
    gsems = (g0, g1, g2, g3)
    ssems = (s0, s1, s2, s3)

    _zero2d(r0, KA, H)
    for q in range(ROWS_T // KA):
        pltpu.async_copy(r0, acc_sh.at[pl.ds(s * ROWS_T + q * KA, KA)], s0)
    pltpu.async_copy(src_hbm.at[pl.ds(base, EW_T)], src_v, s1)
    pltpu.async_copy(ew_hbm.at[pl.ds(base, EW_T)], ew_v, s1)
    pltpu.async_copy(dst_hbm.at[pl.ds(base, EW_T)], dst_v, s1)
    for q in range(ROWS_T // KA):
        pltpu.make_async_copy(r0, acc_sh.at[pl.ds(s * ROWS_T + q * KA, KA)],
                              s0).wait()
    pltpu.make_async_copy(src_hbm.at[pl.ds(base, EW_T)], src_v, s1).wait()
    pltpu.make_async_copy(ew_hbm.at[pl.ds(base, EW_T)], ew_v, s1).wait()
    pltpu.make_async_copy(dst_hbm.at[pl.ds(base, EW_T)], dst_v, s1).wait()
    plsc.subcore_barrier()

    def gsrc(jj):
        return gp_hbm.at[c].at[src_v.at[pl.ds(jj * KA, KA)]]

    def sdst(jj):
        return acc_sh.at[dst_v.at[pl.ds(jj * KA, KA)]]

    def scale(buf, jj):
        def grp(g, _):
            ew16 = ew_v[pl.ds(jj * KA + g * L, L)]
            for t in range(L):
                w16 = _bcast(ew16, t)
                r = g * L + t
                for k in range(H // L):
                    buf[r, pl.ds(k * L, L)] = buf[r, pl.ds(k * L, L)] * w16
            return 0

        lax.fori_loop(0, KA // L, grp, 0)

    def half(X, jj):
        bX = bufs[X]
        Y = (X + 2) % 4
        pltpu.make_async_copy(gsrc(jj), bX, gsems[X]).wait()

        @pl.when(jj >= 2)
        def _():
            pltpu.make_async_copy(bufs[Y], sdst(jj - 2), ssems[Y]).wait()

        @pl.when(jj + 2 < NCA)
        def _():
            pltpu.async_copy(gsrc(jj + 2), bufs[Y], gsems[Y])

        scale(bX, jj)
        pltpu.async_copy(bX, sdst(jj), ssems[X], add=True)

    pltpu.async_copy(gsrc(0), r0, g0)
    pltpu.async_copy(gsrc(1), r1, g1)

    def piped(j4, _):
        for X in range(4):
            half(X, 4 * j4 + X)
        return 0

    lax.fori_loop(0, NCA // 4, piped, 0)
    pltpu.make_async_copy(r2, sdst(NCA - 2), s2).wait()
    pltpu.make_async_copy(r3, sdst(NCA - 1), s3).wait()

    plsc.subcore_barrier()
    pltpu.sync_copy(acc_sh.at[pl.ds(s * ROWS_T, ROWS_T)],
                    agg_hbm.at[c].at[pl.ds(s * ROWS_T, ROWS_T)])


def _agg_kernel(gp, src_p, ew_p, dst_p):
    return pl.kernel(
        _agg_body,
        out_type=jax.ShapeDtypeStruct((NC, NP, H), jnp.float32),
        mesh=_mesh,
        scratch_types=[
            pltpu.VMEM((EW_T,), jnp.int32),
            pltpu.VMEM((EW_T,), jnp.float32),
            pltpu.VMEM((EW_T,), jnp.int32),
            pltpu.VMEM((KA, H), jnp.float32),
            pltpu.VMEM((KA, H), jnp.float32),
            pltpu.VMEM((KA, H), jnp.float32),
            pltpu.VMEM((KA, H), jnp.float32),
            pltpu.VMEM_SHARED((NP, H), jnp.float32),
            pltpu.SemaphoreType.DMA,
            pltpu.SemaphoreType.DMA,
            pltpu.SemaphoreType.DMA,
            pltpu.SemaphoreType.DMA,
            pltpu.SemaphoreType.DMA,
            pltpu.SemaphoreType.DMA,
            pltpu.SemaphoreType.DMA,
            pltpu.SemaphoreType.DMA,
        ],
    )(gp, src_p, ew_p, dst_p)


BR = 2048


def _tcA_body(x_ref, w_ref, du_ref, out_ref):
    deg = 1.0 + du_ref[0, :, 0]
    dinv = lax.rsqrt(deg)
    g = jnp.dot(x_ref[...], w_ref[...], preferred_element_type=jnp.float32)
    gp = g * dinv[:, None]
    out_ref[0] = gp[:, :H]
    out_ref[1] = gp[:, H:]


def _tcA(x, w1, du):
    return pl.pallas_call(
        _tcA_body,
        grid=(NP // BR,),
        in_specs=[
            pl.BlockSpec((BR, D), lambda i: (i, 0)),
            pl.BlockSpec((D, D), lambda i: (0, 0)),
            pl.BlockSpec((NC, BR, H), lambda i: (0, i, 0)),
        ],
        out_specs=pl.BlockSpec((NC, BR, H), lambda i: (0, i, 0)),
        out_shape=jax.ShapeDtypeStruct((NC, NP, H), jnp.float32),
    )(x, w1, du)


def _tcB_body(acc_ref, gp_ref, du_ref, b_ref, w_ref, out_ref):
    dinv = lax.rsqrt(1.0 + du_ref[0, :, 0])
    lo = dinv[:, None] * (acc_ref[0] + gp_ref[0])
    hi = dinv[:, None] * (acc_ref[1] + gp_ref[1])
    h = jnp.concatenate([lo, hi], axis=1) + b_ref[0][None, :]
    h = jnp.maximum(h, 0.0)
    g = jnp.dot(h, w_ref[...], preferred_element_type=jnp.float32)
    gp = g * dinv[:, None]
    out_ref[0] = gp[:, :H]
    out_ref[1] = gp[:, H:]


def _tcB(acc, gp, du, b, w):
    return pl.pallas_call(
        _tcB_body,
        grid=(NP // BR,),
        in_specs=[
            pl.BlockSpec((NC, BR, H), lambda i: (0, i, 0)),
            pl.BlockSpec((NC, BR, H), lambda i: (0, i, 0)),
            pl.BlockSpec((NC, BR, H), lambda i: (0, i, 0)),
            pl.BlockSpec((1, D), lambda i: (0, 0)),
            pl.BlockSpec((D, D), lambda i: (0, 0)),
        ],
        out_specs=pl.BlockSpec((NC, BR, H), lambda i: (0, i, 0)),
        out_shape=jax.ShapeDtypeStruct((NC, NP, H), jnp.float32),
    )(acc, gp, du, b.reshape(1, D), w)


def _tcC_body(acc_ref, gp_ref, du_ref, b_ref, w_ref, out_ref, a3_ref):
    i = pl.program_id(0)
    dinv = lax.rsqrt(1.0 + du_ref[0, :, 0])
    u = du_ref[1, :, 0]
    lo = dinv[:, None] * (acc_ref[0] + gp_ref[0])
    hi = dinv[:, None] * (acc_ref[1] + gp_ref[1])
    h = jnp.concatenate([lo, hi], axis=1) + b_ref[0][None, :]
    h = jnp.maximum(h, 0.0)
    g = jnp.dot(h, w_ref[...], preferred_element_type=jnp.float32)
    gp3 = g * dinv[:, None]
    out_ref[...] = gp3
    part = jnp.dot(u[None, :], gp3, preferred_element_type=jnp.float32)

    @pl.when(i == 0)
    def _():
        a3_ref[...] = jnp.zeros_like(a3_ref)

    a3_ref[...] += part


def _tcC(acc, gp, du, b, w3):
    return pl.pallas_call(
        _tcC_body,
        grid=(NP // BR,),
        in_specs=[
            pl.BlockSpec((NC, BR, H), lambda i: (0, i, 0)),
            pl.BlockSpec((NC, BR, H), lambda i: (0, i, 0)),
            pl.BlockSpec((NC, BR, H), lambda i: (0, i, 0)),
            pl.BlockSpec((1, D), lambda i: (0, 0)),
            pl.BlockSpec((D, C), lambda i: (0, 0)),
        ],
        out_specs=[
            pl.BlockSpec((BR, C), lambda i: (i, 0)),
            pl.BlockSpec((1, C), lambda i: (0, 0)),
        ],
        out_shape=[jax.ShapeDtypeStruct((NP, C), jnp.float32),
                   jax.ShapeDtypeStruct((1, C), jnp.float32)],
    )(acc, gp, du, b.reshape(1, D), w3)


def kernel(x, edge_index, P_vec, W1, b1, W2, b2, W3, b3, index):
    i32 = jnp.int32
    src = edge_index[0].astype(i32)
    dst = edge_index[1].astype(i32)
    pad = EP - E
    src_p = jnp.concatenate([src, jnp.zeros((pad,), i32)])
    dst_p = jnp.concatenate([dst, jnp.zeros((pad,), i32)])
    p_pad = jnp.concatenate([P_vec, jnp.zeros((pad,), jnp.float32)])
    x_p = jnp.concatenate([x, jnp.zeros((NP - N, D), jnp.float32)])
    idx16 = jnp.full((L,), index, i32)

    ew_p, du = _deg_kernel(p_pad, src_p, dst_p, idx16)
    gp1 = _tcA(x_p, W1, du)
    acc1 = _agg_kernel(gp1, src_p, ew_p, dst_p)
    gp2 = _tcB(acc1, gp1, du, b1, W2)
    acc2 = _agg_kernel(gp2, src_p, ew_p, dst_p)
    gp3, acc3 = _tcC(acc2, gp2, du, b2, W3)

    dinv_i = lax.rsqrt(1.0 + du[0, index, 0])
    row = dinv_i * (acc3[0] + gp3[index]) + b3
    m = jnp.max(row)
    logp = row - (m + jnp.log(jnp.sum(jnp.exp(row - m))))
    return logp
